# Initial kernel scaffold; baseline (speedup 1.0000x reference)
import jax
import jax.numpy as jnp
from jax.experimental import pallas as pl


def kernel(concept_inp, relation_inp, concept_table, relation_table):
    raise NotImplementedError("write your pallas kernel here")


# speedup vs baseline: 2.3982x; 2.3982x over previous
import functools

import jax
import jax.numpy as jnp
from jax import lax
from jax.experimental import pallas as pl
from jax.experimental.pallas import tpu as pltpu
from jax.experimental.pallas import tpu_sc as plsc

D = 128
B = 4096 * 50
NC, NS = 2, 16
NW = NC * NS
BPW = B // NW
CH = 128
NCHUNK = BPW // CH

_mesh = plsc.VectorSubcoreMesh(core_axis_name="c", subcore_axis_name="s")


@functools.partial(
    pl.kernel,
    mesh=_mesh,
    out_type=(
        jax.ShapeDtypeStruct((B, D), jnp.float32),
        jax.ShapeDtypeStruct((B, D), jnp.float32),
    ),
    scratch_types=[
        pltpu.VMEM((CH,), jnp.int32),
        pltpu.VMEM((CH, D), jnp.float32),
        pltpu.SemaphoreType.DMA,
    ],
)
def _gather2(cidx_hbm, ridx_hbm, ctab_hbm, rtab_hbm, cout_hbm, rout_hbm,
             idx_v, rows_v, sem):
    wid = lax.axis_index("s") * NC + lax.axis_index("c")
    base = wid * BPW

    def body(i, carry):
        off = base + i * CH
        pltpu.sync_copy(cidx_hbm.at[pl.ds(off, CH)], idx_v)
        pltpu.async_copy(ctab_hbm.at[idx_v], rows_v, sem).wait()
        pltpu.sync_copy(rows_v, cout_hbm.at[pl.ds(off, CH)])
        pltpu.sync_copy(ridx_hbm.at[pl.ds(off, CH)], idx_v)
        pltpu.async_copy(rtab_hbm.at[idx_v], rows_v, sem).wait()
        pltpu.sync_copy(rows_v, rout_hbm.at[pl.ds(off, CH)])
        return carry

    lax.fori_loop(0, NCHUNK, body, 0)


def kernel(concept_inp, relation_inp, concept_table, relation_table):
    shp = concept_inp.shape
    cidx = concept_inp.reshape(-1).astype(jnp.int32)
    ridx = relation_inp.reshape(-1).astype(jnp.int32)
    cout, rout = _gather2(cidx, ridx, concept_table, relation_table)
    return cout.reshape(*shp, D), rout.reshape(*shp, D)


# speedup vs baseline: 2.5017x; 1.0432x over previous
import functools

import jax
import jax.numpy as jnp
from jax import lax
from jax.experimental import pallas as pl
from jax.experimental.pallas import tpu as pltpu
from jax.experimental.pallas import tpu_sc as plsc

D = 128
B = 4096 * 50
NC, NS = 2, 16
NW = NC * NS
BPW = B // NW
CH = 128
NCHUNK = BPW // CH
NBUF = 5
OUTER = NCHUNK // NBUF

_mesh = plsc.VectorSubcoreMesh(core_axis_name="c", subcore_axis_name="s")


@functools.partial(
    pl.kernel,
    mesh=_mesh,
    out_type=(
        jax.ShapeDtypeStruct((B, D), jnp.float32),
        jax.ShapeDtypeStruct((B, D), jnp.float32),
    ),
    scratch_types=(
        [pltpu.VMEM((BPW,), jnp.int32)] * 2
        + [pltpu.VMEM((CH, D), jnp.float32)] * NBUF
        + [pltpu.SemaphoreType.DMA] * (2 * NBUF)
    ),
)
def _gather2(cidx_hbm, ridx_hbm, ctab_hbm, rtab_hbm, cout_hbm, rout_hbm,
             cidx_v, ridx_v, *bufs_and_sems):
    rows = bufs_and_sems[:NBUF]
    gsem = bufs_and_sems[NBUF:2 * NBUF]
    ssem = bufs_and_sems[2 * NBUF:]

    wid = lax.axis_index("s") * NC + lax.axis_index("c")
    base = wid * BPW

    pltpu.sync_copy(cidx_hbm.at[pl.ds(base, BPW)], cidx_v)
    pltpu.sync_copy(ridx_hbm.at[pl.ds(base, BPW)], ridx_v)

    def run_table(idx_v, tab_hbm, out_hbm):
        for b in range(NBUF):
            pltpu.async_copy(tab_hbm.at[idx_v.at[pl.ds(b * CH, CH)]],
                             rows[b], gsem[b])

        def outer(k, carry):
            for b in range(NBUF):
                i = k * NBUF + b
                off = i * CH
                pltpu.make_async_copy(tab_hbm.at[pl.ds(0, CH)],
                                      rows[b], gsem[b]).wait()
                pltpu.async_copy(rows[b], out_hbm.at[pl.ds(base + off, CH)],
                                 ssem[b])
                pltpu.make_async_copy(rows[b], out_hbm.at[pl.ds(0, CH)],
                                      ssem[b]).wait()
                nxt = i + NBUF

                @pl.when(nxt < NCHUNK)
                def _fire():
                    pltpu.async_copy(
                        tab_hbm.at[idx_v.at[pl.ds(nxt * CH, CH)]],
                        rows[b], gsem[b])
            return carry

        lax.fori_loop(0, OUTER, outer, 0)

    run_table(cidx_v, ctab_hbm, cout_hbm)
    run_table(ridx_v, rtab_hbm, rout_hbm)


def kernel(concept_inp, relation_inp, concept_table, relation_table):
    shp = concept_inp.shape
    cidx = concept_inp.reshape(-1).astype(jnp.int32)
    ridx = relation_inp.reshape(-1).astype(jnp.int32)
    cout, rout = _gather2(cidx, ridx, concept_table, relation_table)
    return cout.reshape(*shp, D), rout.reshape(*shp, D)
